# baseline (speedup 1.0000x reference)
import functools

import jax
import jax.numpy as jnp
from jax import lax
from jax.experimental import pallas as pl
from jax.experimental.pallas import tpu as pltpu
from jax.experimental.pallas import tpu_sc as plsc

_STRIDES = (4, 8, 16, 32)
_OUT = 7
_NB = _OUT * _OUT
_C = 256
_CC = _C // 16
_GRP = 7
_NG = _NB // _GRP
_TAPS = 16
_NW = 32


def _splat(ref, pos):
    return plsc.load_gather(ref, [jnp.full((16,), pos, jnp.int32)])


def _make_sc_call(n_rois):
    rois_per_w = (n_rois + _NW - 1) // _NW
    mesh = plsc.VectorSubcoreMesh(core_axis_name="c", subcore_axis_name="s")

    @functools.partial(
        pl.kernel,
        mesh=mesh,
        compiler_params=pltpu.CompilerParams(needs_layout_passes=False),
        out_type=jax.ShapeDtypeStruct((n_rois, _C * _NB), jnp.float32),
        scratch_types=[
            pltpu.VMEM((16,), jnp.float32),
            pltpu.VMEM((2 * _NG, _GRP * _TAPS), jnp.int32),
            pltpu.VMEM((2 * _NB * _TAPS,), jnp.float32),
            pltpu.VMEM((2, _GRP * _TAPS, _C // 2), jnp.uint32),
            pltpu.VMEM((2 * _C * _NB,), jnp.float32),
            pltpu.SemaphoreType.DMA((2,)),
            pltpu.SemaphoreType.DMA((2,)),
        ],
    )
    def call(table, rois, out, roi_v, idx_v, wgt_v, gbuf, obuf, gsem, osem):
        wid = lax.axis_index("s") * 2 + lax.axis_index("c")

        lane = lax.iota(jnp.int32, 16)
        syf = ((lane >> 3) & 1).astype(jnp.float32)
        sxf = ((lane >> 2) & 1).astype(jnp.float32)
        cyb = ((lane >> 1) & 1) == 1
        cxb = (lane & 1) == 1
        cyf = ((lane >> 1) & 1).astype(jnp.float32)
        cxf = (lane & 1).astype(jnp.float32)
        lane2nb = lane * (2 * _NB)

        def do_prep(i):
            n = i * _NW + wid
            ip = i % 2
            pltpu.sync_copy(rois.at[n], roi_v)
            x1 = _splat(roi_v, 1)
            y1 = _splat(roi_v, 2)
            x2 = _splat(roi_v, 3)
            y2 = _splat(roi_v, 4)
            ssv = _splat(roi_v, 5)
            offv = _splat(roi_v, 6)
            wv = _splat(roi_v, 7)
            hv = _splat(roi_v, 8)
            x1s = x1 * ssv
            y1s = y1 * ssv
            bw = jnp.maximum((x2 - x1) * ssv, 1.0) * (1.0 / _OUT)
            bh = jnp.maximum((y2 - y1) * ssv, 1.0) * (1.0 / _OUT)
            wm1 = wv - 1.0
            hm1 = hv - 1.0

            def bin_prep(b, c2):
                py = b // _OUT
                px = b % _OUT
                pyf = py.astype(jnp.float32)
                pxf = px.astype(jnp.float32)
                ys = y1s + (pyf + 0.25 + 0.5 * syf) * bh
                xs = x1s + (pxf + 0.25 + 0.5 * sxf) * bw
                valid = (ys < hv) & (xs < wv)
                yc = jnp.minimum(ys, hm1)
                xc = jnp.minimum(xs, wm1)
                y0f = yc.astype(jnp.int32).astype(jnp.float32)
                x0f = xc.astype(jnp.int32).astype(jnp.float32)
                ly = yc - y0f
                lx = xc - x0f
                yif = jnp.minimum(y0f + cyf, hm1)
                xif = jnp.minimum(x0f + cxf, wm1)
                wy = jnp.where(cyb, ly, 1.0 - ly)
                wx = jnp.where(cxb, lx, 1.0 - lx)
                w = jnp.where(valid, wy * wx * 0.25, 0.0)
                idx = (offv + yif * wv + xif).astype(jnp.int32)
                idx_v[ip * _NG + py, pl.ds(px * _TAPS, _TAPS)] = idx
                wgt_v[pl.ds(ip * (_NB * _TAPS) + b * _TAPS, _TAPS)] = w
                return c2

            lax.fori_loop(0, _NB, bin_prep, 0)

        def gather_copy(i, g):
            gp = (i + g) % 2
            return pltpu.make_async_copy(
                table.at[idx_v.at[(i % 2) * _NG + g]], gbuf.at[gp],
                gsem.at[gp],
            )

        do_prep(0)
        gather_copy(0, 0).start()

        def roi_body(i, carry):
            n = i * _NW + wid

            @pl.when(n < n_rois)
            def _():
                op = i % 2
                obase = op * (_C * _NB)
                wbase = op * (_NB * _TAPS)

                @pl.when(i >= 2)
                def _():
                    pltpu.make_async_copy(
                        obuf.at[pl.ds(obase, _C * _NB)], out.at[n],
                        osem.at[op],
                    ).wait()

                def grp_body(g, c2):
                    gp = (i + g) % 2
                    gather_copy(i, g).wait()

                    @pl.when(g < _NG - 1)
                    def _():
                        gather_copy(i, g + 1).start()

                    @pl.when(g == _NG - 1)
                    def _():
                        nn = (i + 1) * _NW + wid

                        @pl.when(nn < n_rois)
                        def _():
                            do_prep(i + 1)
                            gather_copy(i + 1, 0).start()

                    def bin_comp(j, c3):
                        b = g * _GRP + j
                        base = j * _TAPS
                        wsp = [
                            _splat(wgt_v, wbase + b * _TAPS + t)
                            for t in range(_TAPS)
                        ]
                        for k in range(_C // 32):
                            a0, b0 = plsc.unpack(
                                plsc.bitcast(
                                    gbuf[gp, base, pl.ds(k * 16, 16)],
                                    jnp.bfloat16,
                                ),
                                format=plsc.PackFormat.INTERLEAVED,
                            )
                            acc_a = wsp[0] * a0
                            acc_b = wsp[0] * b0
                            for t in range(1, _TAPS):
                                at, bt = plsc.unpack(
                                    plsc.bitcast(
                                        gbuf[gp, base + t, pl.ds(k * 16, 16)],
                                        jnp.bfloat16,
                                    ),
                                    format=plsc.PackFormat.INTERLEAVED,
                                )
                                acc_a = acc_a + wsp[t] * at
                                acc_b = acc_b + wsp[t] * bt
                            oidx = lane2nb + (obase + (32 * k) * _NB + b)
                            plsc.store_scatter(obuf, [oidx], acc_a)
                            plsc.store_scatter(obuf, [oidx + _NB], acc_b)
                        return c3

                    lax.fori_loop(0, _GRP, bin_comp, 0)
                    return c2

                lax.fori_loop(0, _NG, grp_body, 0)
                pltpu.async_copy(
                    obuf.at[pl.ds(obase, _C * _NB)], out.at[n], osem.at[op]
                )

            return carry

        lax.fori_loop(0, rois_per_w, roi_body, 0)

        for p in range(2):
            pltpu.make_async_copy(
                obuf.at[pl.ds(p * (_C * _NB), _C * _NB)], out.at[wid],
                osem.at[p],
            ).wait()

    return call


def kernel(feats_0, feats_1, feats_2, feats_3, rois):
    feats = [feats_0, feats_1, feats_2, feats_3]
    n = rois.shape[0]

    def _rne16(u):
        return (u + jnp.uint32(0x7FFF) + ((u >> 16) & jnp.uint32(1))) >> 16

    tables = []
    for f in feats:
        u = lax.bitcast_convert_type(f[0], jnp.uint32)
        packed = (_rne16(u[1::2]) << 16) | _rne16(u[0::2])
        tables.append(jnp.transpose(packed, (1, 2, 0)).reshape(-1, _C // 2))
    table = jnp.concatenate(tables, axis=0)

    scale = jnp.sqrt((rois[:, 3] - rois[:, 1]) * (rois[:, 4] - rois[:, 2]))
    lvls = jnp.clip(
        jnp.floor(jnp.log2(scale / 56.0 + 1e-6)), 0, 3
    ).astype(jnp.int32)
    sizes = [feats[i].shape[2] * feats[i].shape[3] for i in range(4)]
    offs_t = jnp.array(
        [0.0, float(sizes[0]), float(sizes[0] + sizes[1]),
         float(sizes[0] + sizes[1] + sizes[2])], jnp.float32)
    dims_t = jnp.array([f.shape[2] for f in feats], jnp.float32)
    ss_t = jnp.array([1.0 / s for s in _STRIDES], jnp.float32)
    zeros = jnp.zeros((n,), jnp.float32)
    rois_aug = jnp.stack(
        [zeros, rois[:, 1], rois[:, 2], rois[:, 3], rois[:, 4],
         ss_t[lvls], offs_t[lvls], dims_t[lvls], dims_t[lvls]]
        + [zeros] * 7,
        axis=1,
    )

    out = _make_sc_call(n)(table, rois_aug)
    return out.reshape(n, _C, _OUT, _OUT)


# speedup vs baseline: 1.0100x; 1.0100x over previous
import functools

import jax
import jax.numpy as jnp
from jax import lax
from jax.experimental import pallas as pl
from jax.experimental.pallas import tpu as pltpu
from jax.experimental.pallas import tpu_sc as plsc

_STRIDES = (4, 8, 16, 32)
_OUT = 7
_NB = _OUT * _OUT
_C = 256
_CC = _C // 16
_GRP = 7
_NG = _NB // _GRP
_TAPS = 16
_NW = 32


def _splat(ref, pos):
    return plsc.load_gather(ref, [jnp.full((16,), pos, jnp.int32)])


_SIZES = (128 * 128, 64 * 64, 32 * 32, 16 * 16)
_OFFS = (0, 16384, 20480, 21504)
_NROWS = 21760
_STRIPE = 128


def _make_pack_call():
    mesh = plsc.VectorSubcoreMesh(core_axis_name="c", subcore_axis_name="s")

    @functools.partial(
        pl.kernel,
        mesh=mesh,
        compiler_params=pltpu.CompilerParams(needs_layout_passes=False),
        out_type=jax.ShapeDtypeStruct((_NROWS, _C // 2), jnp.uint32),
        scratch_types=[
            pltpu.VMEM((_C, _STRIPE), jnp.float32),
            pltpu.VMEM((_STRIPE, _C // 2), jnp.uint32),
        ],
    )
    def call(f0, f1, f2, f3, table, ibuf, sbuf):
        wid = lax.axis_index("s") * 2 + lax.axis_index("c")
        lane = lax.iota(jnp.int32, 16)
        lane2 = lane * 2
        rne_c = jnp.uint32(0x7FFF)
        one = jnp.uint32(1)

        for f, size, off in zip((f0, f1, f2, f3), _SIZES, _OFFS):
            nstripes = size // _STRIPE

            def level_body(si, carry, f=f, nstripes=nstripes, off=off):
                s = si * _NW + wid

                @pl.when(s < nstripes)
                def _():
                    r0 = s * _STRIPE
                    pltpu.sync_copy(f.at[:, pl.ds(r0, _STRIPE)], ibuf)

                    def row_body(r, c2):
                        ridx = jnp.full((16,), r, jnp.int32)
                        for g in range(_C // 32):
                            cidx = lane2 + g * 32
                            lo = plsc.bitcast(
                                plsc.load_gather(ibuf, [cidx, ridx]),
                                jnp.uint32,
                            )
                            hi = plsc.bitcast(
                                plsc.load_gather(ibuf, [cidx + 1, ridx]),
                                jnp.uint32,
                            )
                            lo16 = (lo + rne_c + ((lo >> 16) & one)) >> 16
                            hi16 = (hi + rne_c + ((hi >> 16) & one)) >> 16
                            sbuf[r, pl.ds(g * 16, 16)] = (hi16 << 16) | lo16
                        return c2

                    lax.fori_loop(0, _STRIPE, row_body, 0)
                    pltpu.sync_copy(
                        sbuf, table.at[pl.ds(off + r0, _STRIPE)]
                    )

                return carry

            lax.fori_loop(0, (nstripes + _NW - 1) // _NW, level_body, 0)

    return call


def _make_sc_call(n_rois):
    rois_per_w = (n_rois + _NW - 1) // _NW
    mesh = plsc.VectorSubcoreMesh(core_axis_name="c", subcore_axis_name="s")

    @functools.partial(
        pl.kernel,
        mesh=mesh,
        compiler_params=pltpu.CompilerParams(needs_layout_passes=False),
        out_type=jax.ShapeDtypeStruct((n_rois, _C * _NB), jnp.float32),
        scratch_types=[
            pltpu.VMEM((16,), jnp.float32),
            pltpu.VMEM((2 * _NG, _GRP * _TAPS), jnp.int32),
            pltpu.VMEM((2 * _NB * _TAPS,), jnp.float32),
            pltpu.VMEM((2, _GRP * _TAPS, _C // 2), jnp.uint32),
            pltpu.VMEM((2 * _C * _NB,), jnp.float32),
            pltpu.SemaphoreType.DMA((2,)),
            pltpu.SemaphoreType.DMA((2,)),
        ],
    )
    def call(table, rois, out, roi_v, idx_v, wgt_v, gbuf, obuf, gsem, osem):
        wid = lax.axis_index("s") * 2 + lax.axis_index("c")

        lane = lax.iota(jnp.int32, 16)
        syf = ((lane >> 3) & 1).astype(jnp.float32)
        sxf = ((lane >> 2) & 1).astype(jnp.float32)
        cyb = ((lane >> 1) & 1) == 1
        cxb = (lane & 1) == 1
        cyf = ((lane >> 1) & 1).astype(jnp.float32)
        cxf = (lane & 1).astype(jnp.float32)
        lane2nb = lane * (2 * _NB)

        def do_prep(i):
            n = i * _NW + wid
            ip = i % 2
            pltpu.sync_copy(rois.at[n], roi_v)
            x1 = _splat(roi_v, 1)
            y1 = _splat(roi_v, 2)
            x2 = _splat(roi_v, 3)
            y2 = _splat(roi_v, 4)
            ssv = _splat(roi_v, 5)
            offv = _splat(roi_v, 6)
            wv = _splat(roi_v, 7)
            hv = _splat(roi_v, 8)
            x1s = x1 * ssv
            y1s = y1 * ssv
            bw = jnp.maximum((x2 - x1) * ssv, 1.0) * (1.0 / _OUT)
            bh = jnp.maximum((y2 - y1) * ssv, 1.0) * (1.0 / _OUT)
            wm1 = wv - 1.0
            hm1 = hv - 1.0

            def bin_prep(b, c2):
                py = b // _OUT
                px = b % _OUT
                pyf = py.astype(jnp.float32)
                pxf = px.astype(jnp.float32)
                ys = y1s + (pyf + 0.25 + 0.5 * syf) * bh
                xs = x1s + (pxf + 0.25 + 0.5 * sxf) * bw
                valid = (ys < hv) & (xs < wv)
                yc = jnp.minimum(ys, hm1)
                xc = jnp.minimum(xs, wm1)
                y0f = yc.astype(jnp.int32).astype(jnp.float32)
                x0f = xc.astype(jnp.int32).astype(jnp.float32)
                ly = yc - y0f
                lx = xc - x0f
                yif = jnp.minimum(y0f + cyf, hm1)
                xif = jnp.minimum(x0f + cxf, wm1)
                wy = jnp.where(cyb, ly, 1.0 - ly)
                wx = jnp.where(cxb, lx, 1.0 - lx)
                w = jnp.where(valid, wy * wx * 0.25, 0.0)
                idx = (offv + yif * wv + xif).astype(jnp.int32)
                idx_v[ip * _NG + py, pl.ds(px * _TAPS, _TAPS)] = idx
                wgt_v[pl.ds(ip * (_NB * _TAPS) + b * _TAPS, _TAPS)] = w
                return c2

            lax.fori_loop(0, _NB, bin_prep, 0)

        def gather_copy(i, g):
            gp = (i + g) % 2
            return pltpu.make_async_copy(
                table.at[idx_v.at[(i % 2) * _NG + g]], gbuf.at[gp],
                gsem.at[gp],
            )

        do_prep(0)
        gather_copy(0, 0).start()

        def roi_body(i, carry):
            n = i * _NW + wid

            @pl.when(n < n_rois)
            def _():
                op = i % 2
                obase = op * (_C * _NB)
                wbase = op * (_NB * _TAPS)

                @pl.when(i >= 2)
                def _():
                    pltpu.make_async_copy(
                        obuf.at[pl.ds(obase, _C * _NB)], out.at[n],
                        osem.at[op],
                    ).wait()

                def grp_body(g, c2):
                    gp = (i + g) % 2
                    gather_copy(i, g).wait()

                    @pl.when(g < _NG - 1)
                    def _():
                        gather_copy(i, g + 1).start()

                    @pl.when(g == _NG - 1)
                    def _():
                        nn = (i + 1) * _NW + wid

                        @pl.when(nn < n_rois)
                        def _():
                            do_prep(i + 1)
                            gather_copy(i + 1, 0).start()

                    def bin_comp(j, c3):
                        b = g * _GRP + j
                        base = j * _TAPS
                        wsp = [
                            _splat(wgt_v, wbase + b * _TAPS + t)
                            for t in range(_TAPS)
                        ]
                        for k in range(_C // 32):
                            a0, b0 = plsc.unpack(
                                plsc.bitcast(
                                    gbuf[gp, base, pl.ds(k * 16, 16)],
                                    jnp.bfloat16,
                                ),
                                format=plsc.PackFormat.INTERLEAVED,
                            )
                            acc_a = wsp[0] * a0
                            acc_b = wsp[0] * b0
                            for t in range(1, _TAPS):
                                at, bt = plsc.unpack(
                                    plsc.bitcast(
                                        gbuf[gp, base + t, pl.ds(k * 16, 16)],
                                        jnp.bfloat16,
                                    ),
                                    format=plsc.PackFormat.INTERLEAVED,
                                )
                                acc_a = acc_a + wsp[t] * at
                                acc_b = acc_b + wsp[t] * bt
                            oidx = lane2nb + (obase + (32 * k) * _NB + b)
                            plsc.store_scatter(obuf, [oidx], acc_a)
                            plsc.store_scatter(obuf, [oidx + _NB], acc_b)
                        return c3

                    lax.fori_loop(0, _GRP, bin_comp, 0)
                    return c2

                lax.fori_loop(0, _NG, grp_body, 0)
                pltpu.async_copy(
                    obuf.at[pl.ds(obase, _C * _NB)], out.at[n], osem.at[op]
                )

            return carry

        lax.fori_loop(0, rois_per_w, roi_body, 0)

        for p in range(2):
            pltpu.make_async_copy(
                obuf.at[pl.ds(p * (_C * _NB), _C * _NB)], out.at[wid],
                osem.at[p],
            ).wait()

    return call


def kernel(feats_0, feats_1, feats_2, feats_3, rois):
    feats = [feats_0, feats_1, feats_2, feats_3]
    n = rois.shape[0]

    planes = [f[0].reshape(_C, -1) for f in feats]
    table = _make_pack_call()(*planes)

    scale = jnp.sqrt((rois[:, 3] - rois[:, 1]) * (rois[:, 4] - rois[:, 2]))
    lvls = jnp.clip(
        jnp.floor(jnp.log2(scale / 56.0 + 1e-6)), 0, 3
    ).astype(jnp.int32)
    sizes = [feats[i].shape[2] * feats[i].shape[3] for i in range(4)]
    offs_t = jnp.array(
        [0.0, float(sizes[0]), float(sizes[0] + sizes[1]),
         float(sizes[0] + sizes[1] + sizes[2])], jnp.float32)
    dims_t = jnp.array([f.shape[2] for f in feats], jnp.float32)
    ss_t = jnp.array([1.0 / s for s in _STRIDES], jnp.float32)
    zeros = jnp.zeros((n,), jnp.float32)
    rois_aug = jnp.stack(
        [zeros, rois[:, 1], rois[:, 2], rois[:, 3], rois[:, 4],
         ss_t[lvls], offs_t[lvls], dims_t[lvls], dims_t[lvls]]
        + [zeros] * 7,
        axis=1,
    )

    out = _make_sc_call(n)(table, rois_aug)
    return out.reshape(n, _C, _OUT, _OUT)


# speedup vs baseline: 1.0246x; 1.0144x over previous
import functools

import jax
import jax.numpy as jnp
from jax import lax
from jax.experimental import pallas as pl
from jax.experimental.pallas import tpu as pltpu
from jax.experimental.pallas import tpu_sc as plsc

_STRIDES = (4, 8, 16, 32)
_OUT = 7
_NB = _OUT * _OUT
_C = 256
_CC = _C // 16
_GRP = 7
_NG = _NB // _GRP
_TAPS = 16
_NW = 32


def _splat(ref, pos):
    return plsc.load_gather(ref, [jnp.full((16,), pos, jnp.int32)])


def _make_sc_call(n_rois):
    rois_per_w = (n_rois + _NW - 1) // _NW
    mesh = plsc.VectorSubcoreMesh(core_axis_name="c", subcore_axis_name="s")

    @functools.partial(
        pl.kernel,
        mesh=mesh,
        compiler_params=pltpu.CompilerParams(needs_layout_passes=False),
        out_type=jax.ShapeDtypeStruct((n_rois, _C * _NB), jnp.float32),
        scratch_types=[
            pltpu.VMEM((16,), jnp.float32),
            pltpu.VMEM((2 * _NG, _GRP * _TAPS), jnp.int32),
            pltpu.VMEM((2 * _NB * _TAPS,), jnp.float32),
            pltpu.VMEM((2, _GRP * _TAPS, _C // 2), jnp.float32),
            pltpu.VMEM((2 * _C * _NB,), jnp.float32),
            pltpu.SemaphoreType.DMA((2,)),
            pltpu.SemaphoreType.DMA((2,)),
        ],
    )
    def call(table, rois, out, roi_v, idx_v, wgt_v, gbuf, obuf, gsem, osem):
        wid = lax.axis_index("s") * 2 + lax.axis_index("c")

        lane = lax.iota(jnp.int32, 16)
        syf = ((lane >> 3) & 1).astype(jnp.float32)
        sxf = ((lane >> 2) & 1).astype(jnp.float32)
        cyb = ((lane >> 1) & 1) == 1
        cxb = (lane & 1) == 1
        cyf = ((lane >> 1) & 1).astype(jnp.float32)
        cxf = (lane & 1).astype(jnp.float32)
        lane2nb = lane * (2 * _NB)

        def do_prep(i):
            n = i * _NW + wid
            ip = i % 2
            pltpu.sync_copy(rois.at[n], roi_v)
            x1 = _splat(roi_v, 1)
            y1 = _splat(roi_v, 2)
            x2 = _splat(roi_v, 3)
            y2 = _splat(roi_v, 4)
            ssv = _splat(roi_v, 5)
            offv = _splat(roi_v, 6)
            wv = _splat(roi_v, 7)
            hv = _splat(roi_v, 8)
            x1s = x1 * ssv
            y1s = y1 * ssv
            bw = jnp.maximum((x2 - x1) * ssv, 1.0) * (1.0 / _OUT)
            bh = jnp.maximum((y2 - y1) * ssv, 1.0) * (1.0 / _OUT)
            wm1 = wv - 1.0
            hm1 = hv - 1.0

            def bin_prep(b, c2):
                py = b // _OUT
                px = b % _OUT
                pyf = py.astype(jnp.float32)
                pxf = px.astype(jnp.float32)
                ys = y1s + (pyf + 0.25 + 0.5 * syf) * bh
                xs = x1s + (pxf + 0.25 + 0.5 * sxf) * bw
                valid = (ys < hv) & (xs < wv)
                yc = jnp.minimum(ys, hm1)
                xc = jnp.minimum(xs, wm1)
                y0f = yc.astype(jnp.int32).astype(jnp.float32)
                x0f = xc.astype(jnp.int32).astype(jnp.float32)
                ly = yc - y0f
                lx = xc - x0f
                yif = jnp.minimum(y0f + cyf, hm1)
                xif = jnp.minimum(x0f + cxf, wm1)
                wy = jnp.where(cyb, ly, 1.0 - ly)
                wx = jnp.where(cxb, lx, 1.0 - lx)
                w = jnp.where(valid, wy * wx * 0.25, 0.0)
                idx = (offv + yif * wv + xif).astype(jnp.int32)
                idx_v[ip * _NG + py, pl.ds(px * _TAPS, _TAPS)] = idx
                wgt_v[pl.ds(ip * (_NB * _TAPS) + b * _TAPS, _TAPS)] = w
                return c2

            lax.fori_loop(0, _NB, bin_prep, 0)

        def gather_copy(i, g):
            gp = (i + g) % 2
            return pltpu.make_async_copy(
                table.at[idx_v.at[(i % 2) * _NG + g]], gbuf.at[gp],
                gsem.at[gp],
            )

        do_prep(0)
        gather_copy(0, 0).start()

        def roi_body(i, carry):
            n = i * _NW + wid

            @pl.when(n < n_rois)
            def _():
                op = i % 2
                obase = op * (_C * _NB)
                wbase = op * (_NB * _TAPS)

                @pl.when(i >= 2)
                def _():
                    pltpu.make_async_copy(
                        obuf.at[pl.ds(obase, _C * _NB)], out.at[n],
                        osem.at[op],
                    ).wait()

                def grp_body(g, c2):
                    gp = (i + g) % 2
                    gather_copy(i, g).wait()

                    @pl.when(g < _NG - 1)
                    def _():
                        gather_copy(i, g + 1).start()

                    @pl.when(g == _NG - 1)
                    def _():
                        nn = (i + 1) * _NW + wid

                        @pl.when(nn < n_rois)
                        def _():
                            do_prep(i + 1)
                            gather_copy(i + 1, 0).start()

                    def bin_comp(j, c3):
                        b = g * _GRP + j
                        base = j * _TAPS
                        wsp = [
                            _splat(wgt_v, wbase + b * _TAPS + t)
                            for t in range(_TAPS)
                        ]
                        for k in range(_C // 32):
                            a0, b0 = plsc.unpack(
                                plsc.bitcast(
                                    gbuf[gp, base, pl.ds(k * 16, 16)],
                                    jnp.bfloat16,
                                ),
                                format=plsc.PackFormat.INTERLEAVED,
                            )
                            acc_a = wsp[0] * a0
                            acc_b = wsp[0] * b0
                            for t in range(1, _TAPS):
                                at, bt = plsc.unpack(
                                    plsc.bitcast(
                                        gbuf[gp, base + t, pl.ds(k * 16, 16)],
                                        jnp.bfloat16,
                                    ),
                                    format=plsc.PackFormat.INTERLEAVED,
                                )
                                acc_a = acc_a + wsp[t] * at
                                acc_b = acc_b + wsp[t] * bt
                            oidx = lane2nb + (obase + (32 * k) * _NB + b)
                            plsc.store_scatter(obuf, [oidx], acc_a)
                            plsc.store_scatter(obuf, [oidx + _NB], acc_b)
                        return c3

                    lax.fori_loop(0, _GRP, bin_comp, 0)
                    return c2

                lax.fori_loop(0, _NG, grp_body, 0)
                pltpu.async_copy(
                    obuf.at[pl.ds(obase, _C * _NB)], out.at[n], osem.at[op]
                )

            return carry

        lax.fori_loop(0, rois_per_w, roi_body, 0)

        for p in range(2):
            pltpu.make_async_copy(
                obuf.at[pl.ds(p * (_C * _NB), _C * _NB)], out.at[wid],
                osem.at[p],
            ).wait()

    return call


def kernel(feats_0, feats_1, feats_2, feats_3, rois):
    feats = [feats_0, feats_1, feats_2, feats_3]
    n = rois.shape[0]

    tables = [jnp.transpose(f[0], (1, 2, 0)).reshape(-1, _C) for f in feats]
    table = lax.bitcast_convert_type(
        jnp.concatenate(tables, axis=0)
        .astype(jnp.bfloat16)
        .reshape(-1, _C // 2, 2),
        jnp.float32,
    )

    scale = jnp.sqrt((rois[:, 3] - rois[:, 1]) * (rois[:, 4] - rois[:, 2]))
    lvls = jnp.clip(
        jnp.floor(jnp.log2(scale / 56.0 + 1e-6)), 0, 3
    ).astype(jnp.int32)
    sizes = [feats[i].shape[2] * feats[i].shape[3] for i in range(4)]
    offs_t = jnp.array(
        [0.0, float(sizes[0]), float(sizes[0] + sizes[1]),
         float(sizes[0] + sizes[1] + sizes[2])], jnp.float32)
    dims_t = jnp.array([f.shape[2] for f in feats], jnp.float32)
    ss_t = jnp.array([1.0 / s for s in _STRIDES], jnp.float32)
    zeros = jnp.zeros((n,), jnp.float32)
    rois_aug = jnp.stack(
        [zeros, rois[:, 1], rois[:, 2], rois[:, 3], rois[:, 4],
         ss_t[lvls], offs_t[lvls], dims_t[lvls], dims_t[lvls]]
        + [zeros] * 7,
        axis=1,
    )

    out = _make_sc_call(n)(table, rois_aug)
    return out.reshape(n, _C, _OUT, _OUT)
